# baseline (speedup 1.0000x reference)
import functools

import jax
import jax.numpy as jnp
from jax import lax
from jax.experimental import pallas as pl
from jax.experimental.pallas import tpu as pltpu
from jax.experimental.pallas import tpu_sc as plsc

_K = 20
_KP = 32
_KPW = 128
_R = 256
_CB = 1024
_RE = 256
_RH = 512
_EPS = 1e-5


def _bf(a):
    return a.astype(jnp.bfloat16)


def _mm(a, b):
    return jnp.dot(a, b, preferred_element_type=jnp.float32)


def _bn(y, g, b):
    return g * (y / jnp.sqrt(jnp.float32(1.0 + _EPS))) + b


def _gate_body(x_ref, w1_ref, b1_ref, w2_ref, b2_ref, o_ref):
    xb = x_ref[...]
    z = jnp.maximum(_mm(_bf(xb), w1_ref[...]) + b1_ref[...], 0.0)
    a = jax.nn.sigmoid(_mm(_bf(z), w2_ref[...]) + b2_ref[...])
    o_ref[...] = xb * a


def _gate(xp, w1, b1, w2, b2):
    n, dp = xp.shape
    full = lambda a: pl.BlockSpec(a.shape, lambda i: (0,) * a.ndim)
    return pl.pallas_call(
        _gate_body,
        grid=(n // _RH,),
        in_specs=[
            pl.BlockSpec((_RH, dp), lambda i: (i, 0)),
            full(w1), full(b1), full(w2), full(b2),
        ],
        out_specs=pl.BlockSpec((_RH, dp), lambda i: (i, 0)),
        out_shape=jax.ShapeDtypeStruct((n, dp), jnp.float32),
    )(xp, w1, b1, w2, b2)


def _knn_body(bounds_ref, xr_ref, sqr_ref, br_ref, xtc_ref, sqc_ref, bc_ref,
              idx_ref, val_ref, *, n_total):
    g = pl.program_id(0)
    r = xr_ref.shape[0]
    cb_sz = xtc_ref.shape[2]
    inf = jnp.float32(jnp.inf)
    xr = xr_ref[...]
    sqr = sqr_ref[...]
    br = br_ref[...]
    lane_k = lax.broadcasted_iota(jnp.int32, (r, _KP), 1)
    bv0 = jnp.full((r, _KP), inf, jnp.float32)
    bi0 = jnp.full((r, _KP), n_total, jnp.int32)
    ci = lax.broadcasted_iota(jnp.int32, (cb_sz, 1), 0)
    cols = jnp.concatenate(
        [(ci // 32).astype(jnp.float32), (ci % 32).astype(jnp.float32),
         jnp.ones((cb_sz, 1), jnp.float32)], axis=1)

    def blk(cb, carry):
        bv, bi = carry
        xc = xtc_ref[cb]
        sqc = sqc_ref[cb]
        bc = bc_ref[cb]
        d0 = (sqr + sqc) - 2.0 * _mm(xr, xc)
        d0 = jnp.where(br != bc, inf, d0)
        base = cb * cb_sz

        e, lv, li = d0, bv0, bi0
        tie = jnp.zeros((r, 1), jnp.float32)
        for t in range(_K):
            m = jnp.min(e, axis=1, keepdims=True)
            oh = e == m
            res = _mm(jnp.where(oh, 1.0, 0.0), cols)
            am = (res[:, 0:1] * 32.0 + res[:, 1:2]).astype(jnp.int32) + base
            tie = jnp.maximum(tie, jnp.where(m < inf, res[:, 2:3], 0.0))
            e = jnp.where(oh, inf, e)
            sel = lane_k == t
            lv = jnp.where(sel, m, lv)
            li = jnp.where(sel, am, li)

        def exact(_):
            iv = base + lax.broadcasted_iota(jnp.int32, (r, cb_sz), 1)
            e2, lv2, li2 = d0, bv0, bi0
            for t in range(_K):
                m = jnp.min(e2, axis=1, keepdims=True)
                am = jnp.min(jnp.where(e2 == m, iv, n_total), axis=1,
                             keepdims=True)
                sel = lane_k == t
                lv2 = jnp.where(sel, m, lv2)
                li2 = jnp.where(sel, am, li2)
                e2 = jnp.where(iv == am, inf, e2)
            return lv2, li2

        lv, li = lax.cond(jnp.max(tie) > 1.5, exact,
                          lambda _: (lv, li), 0)

        ev = jnp.concatenate([bv, lv], axis=1)
        ei = jnp.concatenate([bi, li], axis=1)
        nbv, nbi = bv0, bi0
        for t in range(_K):
            m = jnp.min(ev, axis=1, keepdims=True)
            am = jnp.min(jnp.where(ev == m, ei, n_total), axis=1,
                         keepdims=True)
            sel = lane_k == t
            nbv = jnp.where(sel, m, nbv)
            nbi = jnp.where(sel, am, nbi)
            ev = jnp.where(ei == am, inf, ev)
        return nbv, nbi

    bv, bi = lax.fori_loop(bounds_ref[g, 0], bounds_ref[g, 1], blk, (bv0, bi0))
    idx_ref[...] = bi
    val_ref[...] = bv


def _knn(featb, sq, brow, bcol, bounds):
    n, dp = featb.shape
    ncb = n // _CB
    xtc = featb.T.reshape(dp, ncb, _CB).transpose(1, 0, 2)
    sqc = sq.reshape(ncb, 1, _CB)
    sqr = sq[:, None]
    grid_spec = pltpu.PrefetchScalarGridSpec(
        num_scalar_prefetch=1,
        grid=(n // _R,),
        in_specs=[
            pl.BlockSpec((_R, dp), lambda g, b: (g, 0)),
            pl.BlockSpec((_R, 1), lambda g, b: (g, 0)),
            pl.BlockSpec((_R, 1), lambda g, b: (g, 0)),
            pl.BlockSpec((ncb, dp, _CB), lambda g, b: (0, 0, 0)),
            pl.BlockSpec((ncb, 1, _CB), lambda g, b: (0, 0, 0)),
            pl.BlockSpec((ncb, 1, _CB), lambda g, b: (0, 0, 0)),
        ],
        out_specs=[
            pl.BlockSpec((_R, _KP), lambda g, b: (g, 0)),
            pl.BlockSpec((_R, _KP), lambda g, b: (g, 0)),
        ],
    )
    return pl.pallas_call(
        functools.partial(_knn_body, n_total=n),
        grid_spec=grid_spec,
        out_shape=[
            jax.ShapeDtypeStruct((n, _KP), jnp.int32),
            jax.ShapeDtypeStruct((n, _KP), jnp.float32),
        ],
    )(bounds, featb, sqr, brow, xtc, sqc, bcol)


def _gather_rows(table, idx):
    m = idx.shape[0]
    d = table.shape[1]
    info = plsc.get_sparse_core_info()
    nw = info.num_cores * info.num_subcores
    per_w = m // nw
    chunk = 512
    steps = per_w // chunk
    mesh = plsc.VectorSubcoreMesh(core_axis_name="c", subcore_axis_name="s")

    @functools.partial(
        pl.kernel,
        out_type=jax.ShapeDtypeStruct((m, d), jnp.float32),
        mesh=mesh,
        compiler_params=pltpu.CompilerParams(use_tc_tiling_on_sc=False),
        scratch_types=[
            pltpu.VMEM((chunk,), jnp.int32),
            pltpu.VMEM((chunk, d), jnp.float32),
            pltpu.SemaphoreType.DMA,
        ],
    )
    def gk(table_hbm, idx_hbm, out_hbm, idx_v, rows_v, sem):
        wid = lax.axis_index("s") * info.num_cores + lax.axis_index("c")
        base = wid * per_w

        def body(i, carry):
            off = base + i * chunk
            pltpu.sync_copy(idx_hbm.at[pl.ds(off, chunk)], idx_v)
            pltpu.async_copy(table_hbm.at[idx_v], rows_v, sem).wait()
            pltpu.sync_copy(rows_v, out_hbm.at[pl.ds(off, chunk)])
            return carry

        lax.fori_loop(0, steps, body, 0)

    return gk(table, idx)


def _edge_body(x_ref, g_ref, wa_ref, wb_ref, b1_ref, g1_ref, bb1_ref,
               w2_ref, b2_ref, g2_ref, bb2_ref, o_ref):
    x = x_ref[...]
    s = _mm(_bf(x), wa_ref[...])
    b1 = b1_ref[...]
    g1 = g1_ref[...]
    bb1 = bb1_ref[...]
    wb = wb_ref[...]
    w2 = w2_ref[...]
    b2 = b2_ref[...]
    g2 = g2_ref[...]
    bb2 = bb2_ref[...]
    acc = jnp.zeros(o_ref.shape, jnp.float32)
    for j in range(_K):
        t = _mm(_bf(g_ref[j] - x), wb)
        h1 = jnp.maximum(_bn((s + t) + b1, g1, bb1), 0.0)
        h2 = jnp.maximum(_bn(_mm(_bf(h1), w2) + b2, g2, bb2), 0.0)
        acc = jnp.maximum(acc, h2)
    o_ref[...] = acc


def _edge(feat, gath, wa, wb, b1, g1, bb1, w2, b2, g2, bb2):
    n, din = feat.shape
    dout = w2.shape[1]
    full = lambda a: pl.BlockSpec(a.shape, lambda i: (0,) * a.ndim)
    return pl.pallas_call(
        _edge_body,
        grid=(n // _RE,),
        in_specs=[
            pl.BlockSpec((_RE, din), lambda i: (i, 0)),
            pl.BlockSpec((_K, _RE, din), lambda i: (0, i, 0)),
            full(wa), full(wb), full(b1), full(g1), full(bb1),
            full(w2), full(b2), full(g2), full(bb2),
        ],
        out_specs=pl.BlockSpec((_RE, dout), lambda i: (i, 0)),
        out_shape=jax.ShapeDtypeStruct((n, dout), jnp.float32),
    )(feat, gath, wa, wb, b1, g1, bb1, w2, b2, g2, bb2)


def _conv(feat, sq, brow, bcol, bounds, start_r, size_r, p, tag, din_pad):
    n, dp = feat.shape
    w1 = p[tag + 'W1']
    din = w1.shape[1] // 2
    pad = ((0, din_pad - din), (0, 0))
    wa = _bf(jnp.pad(w1[:, :din].T, pad))
    wb = _bf(jnp.pad(w1[:, din:].T, pad))
    idx, vals = _knn(_bf(feat), sq, brow, bcol, bounds)
    idx = idx[:, :_K]
    valid = jnp.isfinite(vals[:, :_K])
    cnt = jnp.sum(valid, axis=1).astype(jnp.int32)
    pfill = jnp.arange(_K, dtype=jnp.int32)[None, :] - cnt[:, None]
    fill = jnp.where(pfill < start_r[:, None], pfill, pfill + size_r[:, None])
    idx = jnp.where(valid, idx, fill)
    idx_t = idx.T.reshape(-1)
    gath = _gather_rows(feat, idx_t).reshape(_K, n, dp)
    row = lambda v: v[None, :]
    return _edge(feat, gath, wa, wb,
                 row(p[tag + 'b1']), row(p[tag + 'g1']), row(p[tag + 'bb1']),
                 _bf(p[tag + 'W2'].T),
                 row(p[tag + 'b2']), row(p[tag + 'g2']), row(p[tag + 'bb2']))


def _head_body(x1_ref, x2_ref, x3_ref, w1a_ref, w1b_ref, w1c_ref, b1_ref,
               g1_ref, bb1_ref, w2_ref, b2_ref, g2_ref, bb2_ref,
               w3_ref, b3_ref, o_ref):
    y = (_mm(_bf(x1_ref[...]), w1a_ref[...])
         + _mm(_bf(x2_ref[...]), w1b_ref[...])
         + _mm(_bf(x3_ref[...]), w1c_ref[...])
         + b1_ref[...])
    z = jnp.maximum(_bn(y, g1_ref[...], bb1_ref[...]), 0.0)
    z = jnp.maximum(
        _bn(_mm(_bf(z), w2_ref[...]) + b2_ref[...], g2_ref[...], bb2_ref[...]),
        0.0)
    o_ref[...] = _mm(_bf(z), w3_ref[...]) + b3_ref[...]


def _head(x1, x2, x3, *ws):
    n = x1.shape[0]
    full = lambda a: pl.BlockSpec(a.shape, lambda i: (0,) * a.ndim)
    return pl.pallas_call(
        _head_body,
        grid=(n // _RH,),
        in_specs=[
            pl.BlockSpec((_RH, x1.shape[1]), lambda i: (i, 0)),
            pl.BlockSpec((_RH, x2.shape[1]), lambda i: (i, 0)),
            pl.BlockSpec((_RH, x3.shape[1]), lambda i: (i, 0)),
        ] + [full(w) for w in ws],
        out_specs=pl.BlockSpec((_RH, 8), lambda i: (i, 0)),
        out_shape=jax.ShapeDtypeStruct((n, 8), jnp.float32),
    )(x1, x2, x3, *ws)


def kernel(x, batch, params):
    p = params
    n, d_in = x.shape
    batch = batch.astype(jnp.int32)

    xp = jnp.pad(x, ((0, 0), (0, 16 - d_in)))
    w1g = _bf(jnp.pad(p['aW1'].T, ((0, 16 - d_in), (0, 0))))
    w2g = _bf(jnp.pad(p['aW2'].T, ((0, 0), (0, 16 - d_in))))
    b2g = jnp.pad(p['ab2'], (0, 16 - d_in))[None, :]
    x0 = _gate(xp, w1g, p['ab1'][None, :], w2g, b2g)

    ii = jnp.arange(n, dtype=jnp.int32)
    first = jnp.concatenate([jnp.ones((1,), jnp.bool_),
                             batch[1:] != batch[:-1]])
    last = jnp.concatenate([batch[1:] != batch[:-1],
                            jnp.ones((1,), jnp.bool_)])
    start_r = lax.associative_scan(jnp.maximum, jnp.where(first, ii, 0))
    end_r = lax.associative_scan(jnp.minimum, jnp.where(last, ii + 1, n),
                                 reverse=True)
    size_r = end_r - start_r
    r0 = jnp.arange(0, n, _R, dtype=jnp.int32)
    lo = start_r[r0]
    hi = end_r[r0 + _R - 1]
    bounds = jnp.stack([lo // _CB, (hi + _CB - 1) // _CB], axis=1)
    brow = batch[:, None]
    bcol = batch.reshape(n // _CB, 1, _CB)

    sq0 = jnp.sum(x0[:, :d_in] * x0[:, :d_in], axis=1)
    x1 = _conv(x0, sq0, brow, bcol, bounds, start_r, size_r, p, 'c1', 16)
    sq1 = jnp.sum(x1 * x1, axis=1)
    x2 = _conv(x1, sq1, brow, bcol, bounds, start_r, size_r, p, 'c2', 64)
    sq2 = jnp.sum(x2 * x2, axis=1)
    x3 = _conv(x2, sq2, brow, bcol, bounds, start_r, size_r, p, 'c3', 128)

    w1t = _bf(p['pW1'].T)
    row = lambda v: v[None, :]
    out = _head(x1, x2, x3,
                w1t[:64], w1t[64:192], w1t[192:],
                row(p['pb1']), row(p['pg1']), row(p['pbb1']),
                _bf(p['pW2'].T), row(p['pb2']), row(p['pg2']), row(p['pbb2']),
                _bf(jnp.pad(p['pW3'].T, ((0, 0), (0, 3)))),
                row(jnp.pad(p['pb3'], (0, 3))))
    return out[:, :5]


# speedup vs baseline: 1.2390x; 1.2390x over previous
import functools

import jax
import jax.numpy as jnp
from jax import lax
from jax.experimental import pallas as pl
from jax.experimental.pallas import tpu as pltpu
from jax.experimental.pallas import tpu_sc as plsc

_K = 20
_KP = 32
_KPW = 128
_R = 256
_CB = 512
_RE = 256
_RH = 512
_EPS = 1e-5


def _bf(a):
    return a.astype(jnp.bfloat16)


def _mm(a, b):
    return jnp.dot(a, b, preferred_element_type=jnp.float32)


def _bn(y, g, b):
    return g * (y / jnp.sqrt(jnp.float32(1.0 + _EPS))) + b


def _gate_body(x_ref, w1_ref, b1_ref, w2_ref, b2_ref, o_ref):
    xb = x_ref[...]
    z = jnp.maximum(_mm(_bf(xb), w1_ref[...]) + b1_ref[...], 0.0)
    a = jax.nn.sigmoid(_mm(_bf(z), w2_ref[...]) + b2_ref[...])
    o_ref[...] = xb * a


def _gate(xp, w1, b1, w2, b2):
    n, dp = xp.shape
    full = lambda a: pl.BlockSpec(a.shape, lambda i: (0,) * a.ndim)
    return pl.pallas_call(
        _gate_body,
        grid=(n // _RH,),
        in_specs=[
            pl.BlockSpec((_RH, dp), lambda i: (i, 0)),
            full(w1), full(b1), full(w2), full(b2),
        ],
        out_specs=pl.BlockSpec((_RH, dp), lambda i: (i, 0)),
        out_shape=jax.ShapeDtypeStruct((n, dp), jnp.float32),
    )(xp, w1, b1, w2, b2)


def _knn_body(bounds_ref, xr_ref, sqr_ref, br_ref, xtc_ref, sqc_ref, bc_ref,
              idx_ref, *, n_total):
    g = pl.program_id(0)
    r = xr_ref.shape[0]
    cb_sz = xtc_ref.shape[2]
    inf = jnp.float32(jnp.inf)
    xr = xr_ref[...]
    sqr = sqr_ref[...]
    br = br_ref[...]
    lane_k = lax.broadcasted_iota(jnp.int32, (r, _KPW), 1)
    bv0 = jnp.full((r, _KPW), inf, jnp.float32)
    bi0 = jnp.full((r, _KPW), n_total, jnp.int32)

    def blk(cb, carry):
        bv, bi = carry
        xc = xtc_ref[cb]
        sqc = sqc_ref[cb]
        bc = bc_ref[cb]
        d = (sqr + sqc) - 2.0 * _mm(xr, xc)
        masked = br != bc
        d = jnp.where(masked, inf, d)
        ic = jnp.where(
            masked, n_total,
            cb * cb_sz + lax.broadcasted_iota(jnp.int32, (r, cb_sz), 1))
        e = jnp.concatenate([d, bv], axis=1)
        iv = jnp.concatenate([ic, bi], axis=1)
        nbv, nbi = bv0, bi0
        for t in range(_K):
            m = jnp.min(e, axis=1, keepdims=True)
            am = jnp.min(jnp.where(e == m, iv, n_total), axis=1, keepdims=True)
            nbv = jnp.where(lane_k == t, m, nbv)
            nbi = jnp.where(lane_k == t, am, nbi)
            e = jnp.where(iv == am, inf, e)
        return nbv, nbi

    _, bi = lax.fori_loop(bounds_ref[g, 0], bounds_ref[g, 1], blk, (bv0, bi0))
    idx_ref[...] = bi[:, :_KP]


def _knn(featb, sq, brow, bcol, bounds):
    n, dp = featb.shape
    ncb = n // _CB
    xtc = featb.T.reshape(dp, ncb, _CB).transpose(1, 0, 2)
    sqc = sq.reshape(ncb, 1, _CB)
    sqr = sq[:, None]
    grid_spec = pltpu.PrefetchScalarGridSpec(
        num_scalar_prefetch=1,
        grid=(n // _R,),
        in_specs=[
            pl.BlockSpec((_R, dp), lambda g, b: (g, 0)),
            pl.BlockSpec((_R, 1), lambda g, b: (g, 0)),
            pl.BlockSpec((_R, 1), lambda g, b: (g, 0)),
            pl.BlockSpec((ncb, dp, _CB), lambda g, b: (0, 0, 0)),
            pl.BlockSpec((ncb, 1, _CB), lambda g, b: (0, 0, 0)),
            pl.BlockSpec((ncb, 1, _CB), lambda g, b: (0, 0, 0)),
        ],
        out_specs=pl.BlockSpec((_R, _KP), lambda g, b: (g, 0)),
    )
    return pl.pallas_call(
        functools.partial(_knn_body, n_total=n),
        grid_spec=grid_spec,
        out_shape=jax.ShapeDtypeStruct((n, _KP), jnp.int32),
    )(bounds, featb, sqr, brow, xtc, sqc, bcol)


def _gather_rows(table, idx):
    m = idx.shape[0]
    d = table.shape[1]
    info = plsc.get_sparse_core_info()
    nw = info.num_cores * info.num_subcores
    per_w = m // nw
    chunk = 512
    steps = per_w // chunk
    mesh = plsc.VectorSubcoreMesh(core_axis_name="c", subcore_axis_name="s")

    @functools.partial(
        pl.kernel,
        out_type=jax.ShapeDtypeStruct((m, d), jnp.float32),
        mesh=mesh,
        compiler_params=pltpu.CompilerParams(use_tc_tiling_on_sc=False),
        scratch_types=[
            pltpu.VMEM((chunk,), jnp.int32),
            pltpu.VMEM((chunk, d), jnp.float32),
            pltpu.SemaphoreType.DMA,
        ],
    )
    def gk(table_hbm, idx_hbm, out_hbm, idx_v, rows_v, sem):
        wid = lax.axis_index("s") * info.num_cores + lax.axis_index("c")
        base = wid * per_w

        def body(i, carry):
            off = base + i * chunk
            pltpu.sync_copy(idx_hbm.at[pl.ds(off, chunk)], idx_v)
            pltpu.async_copy(table_hbm.at[idx_v], rows_v, sem).wait()
            pltpu.sync_copy(rows_v, out_hbm.at[pl.ds(off, chunk)])
            return carry

        lax.fori_loop(0, steps, body, 0)

    return gk(table, idx)


def _edge_body(x_ref, g_ref, wa_ref, wb_ref, b1_ref, g1_ref, bb1_ref,
               w2_ref, b2_ref, g2_ref, bb2_ref, o_ref):
    x = x_ref[...]
    s = _mm(_bf(x), wa_ref[...])
    b1 = b1_ref[...]
    g1 = g1_ref[...]
    bb1 = bb1_ref[...]
    wb = wb_ref[...]
    w2 = w2_ref[...]
    b2 = b2_ref[...]
    g2 = g2_ref[...]
    bb2 = bb2_ref[...]
    acc = jnp.zeros(o_ref.shape, jnp.float32)
    for j in range(_K):
        t = _mm(_bf(g_ref[j] - x), wb)
        h1 = jnp.maximum(_bn((s + t) + b1, g1, bb1), 0.0)
        h2 = jnp.maximum(_bn(_mm(_bf(h1), w2) + b2, g2, bb2), 0.0)
        acc = jnp.maximum(acc, h2)
    o_ref[...] = acc


def _edge(feat, gath, wa, wb, b1, g1, bb1, w2, b2, g2, bb2):
    n, din = feat.shape
    dout = w2.shape[1]
    full = lambda a: pl.BlockSpec(a.shape, lambda i: (0,) * a.ndim)
    return pl.pallas_call(
        _edge_body,
        grid=(n // _RE,),
        in_specs=[
            pl.BlockSpec((_RE, din), lambda i: (i, 0)),
            pl.BlockSpec((_K, _RE, din), lambda i: (0, i, 0)),
            full(wa), full(wb), full(b1), full(g1), full(bb1),
            full(w2), full(b2), full(g2), full(bb2),
        ],
        out_specs=pl.BlockSpec((_RE, dout), lambda i: (i, 0)),
        out_shape=jax.ShapeDtypeStruct((n, dout), jnp.float32),
    )(feat, gath, wa, wb, b1, g1, bb1, w2, b2, g2, bb2)


def _conv(feat, sq, brow, bcol, bounds, start_r, size_r, p, tag, din_pad):
    n, dp = feat.shape
    w1 = p[tag + 'W1']
    din = w1.shape[1] // 2
    pad = ((0, din_pad - din), (0, 0))
    wa = _bf(jnp.pad(w1[:, :din].T, pad))
    wb = _bf(jnp.pad(w1[:, din:].T, pad))
    idx = _knn(_bf(feat), sq, brow, bcol, bounds)[:, :_K]
    valid = idx < n
    cnt = jnp.sum(valid, axis=1).astype(jnp.int32)
    pfill = jnp.arange(_K, dtype=jnp.int32)[None, :] - cnt[:, None]
    fill = jnp.where(pfill < start_r[:, None], pfill, pfill + size_r[:, None])
    idx = jnp.where(valid, idx, fill)
    idx_t = idx.T.reshape(-1)
    gath = _gather_rows(feat, idx_t).reshape(_K, n, dp)
    row = lambda v: v[None, :]
    return _edge(feat, gath, wa, wb,
                 row(p[tag + 'b1']), row(p[tag + 'g1']), row(p[tag + 'bb1']),
                 _bf(p[tag + 'W2'].T),
                 row(p[tag + 'b2']), row(p[tag + 'g2']), row(p[tag + 'bb2']))


def _head_body(x1_ref, x2_ref, x3_ref, w1a_ref, w1b_ref, w1c_ref, b1_ref,
               g1_ref, bb1_ref, w2_ref, b2_ref, g2_ref, bb2_ref,
               w3_ref, b3_ref, o_ref):
    y = (_mm(_bf(x1_ref[...]), w1a_ref[...])
         + _mm(_bf(x2_ref[...]), w1b_ref[...])
         + _mm(_bf(x3_ref[...]), w1c_ref[...])
         + b1_ref[...])
    z = jnp.maximum(_bn(y, g1_ref[...], bb1_ref[...]), 0.0)
    z = jnp.maximum(
        _bn(_mm(_bf(z), w2_ref[...]) + b2_ref[...], g2_ref[...], bb2_ref[...]),
        0.0)
    o_ref[...] = _mm(_bf(z), w3_ref[...]) + b3_ref[...]


def _head(x1, x2, x3, *ws):
    n = x1.shape[0]
    full = lambda a: pl.BlockSpec(a.shape, lambda i: (0,) * a.ndim)
    return pl.pallas_call(
        _head_body,
        grid=(n // _RH,),
        in_specs=[
            pl.BlockSpec((_RH, x1.shape[1]), lambda i: (i, 0)),
            pl.BlockSpec((_RH, x2.shape[1]), lambda i: (i, 0)),
            pl.BlockSpec((_RH, x3.shape[1]), lambda i: (i, 0)),
        ] + [full(w) for w in ws],
        out_specs=pl.BlockSpec((_RH, 8), lambda i: (i, 0)),
        out_shape=jax.ShapeDtypeStruct((n, 8), jnp.float32),
    )(x1, x2, x3, *ws)


def kernel(x, batch, params):
    p = params
    n, d_in = x.shape
    batch = batch.astype(jnp.int32)

    xp = jnp.pad(x, ((0, 0), (0, 16 - d_in)))
    w1g = _bf(jnp.pad(p['aW1'].T, ((0, 16 - d_in), (0, 0))))
    w2g = _bf(jnp.pad(p['aW2'].T, ((0, 0), (0, 16 - d_in))))
    b2g = jnp.pad(p['ab2'], (0, 16 - d_in))[None, :]
    x0 = _gate(xp, w1g, p['ab1'][None, :], w2g, b2g)

    ii = jnp.arange(n, dtype=jnp.int32)
    first = jnp.concatenate([jnp.ones((1,), jnp.bool_),
                             batch[1:] != batch[:-1]])
    last = jnp.concatenate([batch[1:] != batch[:-1],
                            jnp.ones((1,), jnp.bool_)])
    start_r = lax.associative_scan(jnp.maximum, jnp.where(first, ii, 0))
    end_r = lax.associative_scan(jnp.minimum, jnp.where(last, ii + 1, n),
                                 reverse=True)
    size_r = end_r - start_r
    r0 = jnp.arange(0, n, _R, dtype=jnp.int32)
    lo = start_r[r0]
    hi = end_r[r0 + _R - 1]
    bounds = jnp.stack([lo // _CB, (hi + _CB - 1) // _CB], axis=1)
    brow = batch[:, None]
    bcol = batch.reshape(n // _CB, 1, _CB)

    sq0 = jnp.sum(x0[:, :d_in] * x0[:, :d_in], axis=1)
    x1 = _conv(x0, sq0, brow, bcol, bounds, start_r, size_r, p, 'c1', 16)
    sq1 = jnp.sum(x1 * x1, axis=1)
    x2 = _conv(x1, sq1, brow, bcol, bounds, start_r, size_r, p, 'c2', 64)
    sq2 = jnp.sum(x2 * x2, axis=1)
    x3 = _conv(x2, sq2, brow, bcol, bounds, start_r, size_r, p, 'c3', 128)

    w1t = _bf(p['pW1'].T)
    row = lambda v: v[None, :]
    out = _head(x1, x2, x3,
                w1t[:64], w1t[64:192], w1t[192:],
                row(p['pb1']), row(p['pg1']), row(p['pbb1']),
                _bf(p['pW2'].T), row(p['pb2']), row(p['pg2']), row(p['pbb2']),
                _bf(jnp.pad(p['pW3'].T, ((0, 0), (0, 3)))),
                row(jnp.pad(p['pb3'], (0, 3))))
    return out[:, :5]


# speedup vs baseline: 1.5495x; 1.2506x over previous
import functools

import jax
import jax.numpy as jnp
from jax import lax
from jax.experimental import pallas as pl
from jax.experimental.pallas import tpu as pltpu
from jax.experimental.pallas import tpu_sc as plsc

_K = 20
_KP = 32
_KPW = 128
_R = 512
_CB = 1024
_RE = 256
_RH = 512
_EPS = 1e-5


def _bf(a):
    return a.astype(jnp.bfloat16)


def _mm(a, b):
    return jnp.dot(a, b, preferred_element_type=jnp.float32)


def _bn(y, g, b):
    return g * (y / jnp.sqrt(jnp.float32(1.0 + _EPS))) + b


def _gate_body(x_ref, w1_ref, b1_ref, w2_ref, b2_ref, o_ref):
    xb = x_ref[...]
    z = jnp.maximum(_mm(_bf(xb), w1_ref[...]) + b1_ref[...], 0.0)
    a = jax.nn.sigmoid(_mm(_bf(z), w2_ref[...]) + b2_ref[...])
    o_ref[...] = xb * a


def _gate(xp, w1, b1, w2, b2):
    n, dp = xp.shape
    full = lambda a: pl.BlockSpec(a.shape, lambda i: (0,) * a.ndim)
    return pl.pallas_call(
        _gate_body,
        grid=(n // _RH,),
        in_specs=[
            pl.BlockSpec((_RH, dp), lambda i: (i, 0)),
            full(w1), full(b1), full(w2), full(b2),
        ],
        out_specs=pl.BlockSpec((_RH, dp), lambda i: (i, 0)),
        out_shape=jax.ShapeDtypeStruct((n, dp), jnp.float32),
    )(xp, w1, b1, w2, b2)


def _knn_body(bounds_ref, xr_ref, sqr_ref, br_ref, xtc_ref, sqc_ref, bc_ref,
              idx_ref, *, n_total):
    g = pl.program_id(0)
    r = xr_ref.shape[0]
    cb_sz = xtc_ref.shape[2]
    inf = jnp.float32(jnp.inf)
    xr = xr_ref[...]
    sqr = sqr_ref[...]
    br = br_ref[...]
    lane_k = lax.broadcasted_iota(jnp.int32, (r, _KPW), 1)
    bv0 = jnp.full((r, _KPW), inf, jnp.float32)
    bi0 = jnp.full((r, _KPW), n_total, jnp.int32)

    def blk(cb, carry):
        bv, bi = carry
        xc = xtc_ref[cb]
        sqc = sqc_ref[cb]
        bc = bc_ref[cb]
        d = (sqr + sqc) - 2.0 * _mm(xr, xc)
        masked = br != bc
        d = jnp.where(masked, inf, d)
        ic = jnp.where(
            masked, n_total,
            cb * cb_sz + lax.broadcasted_iota(jnp.int32, (r, cb_sz), 1))
        e = jnp.concatenate([d, bv], axis=1)
        iv = jnp.concatenate([ic, bi], axis=1)
        nbv, nbi = bv0, bi0
        for t in range(_K):
            m = jnp.min(e, axis=1, keepdims=True)
            am = jnp.min(jnp.where(e == m, iv, n_total), axis=1, keepdims=True)
            nbv = jnp.where(lane_k == t, m, nbv)
            nbi = jnp.where(lane_k == t, am, nbi)
            e = jnp.where(iv == am, inf, e)
        return nbv, nbi

    _, bi = lax.fori_loop(bounds_ref[g, 0], bounds_ref[g, 1], blk, (bv0, bi0))
    idx_ref[...] = bi[:, :_KP]


def _knn(featb, sq, brow, bcol, bounds):
    n, dp = featb.shape
    ncb = n // _CB
    xtc = featb.T.reshape(dp, ncb, _CB).transpose(1, 0, 2)
    sqc = sq.reshape(ncb, 1, _CB)
    sqr = sq[:, None]
    grid_spec = pltpu.PrefetchScalarGridSpec(
        num_scalar_prefetch=1,
        grid=(n // _R,),
        in_specs=[
            pl.BlockSpec((_R, dp), lambda g, b: (g, 0)),
            pl.BlockSpec((_R, 1), lambda g, b: (g, 0)),
            pl.BlockSpec((_R, 1), lambda g, b: (g, 0)),
            pl.BlockSpec((ncb, dp, _CB), lambda g, b: (0, 0, 0)),
            pl.BlockSpec((ncb, 1, _CB), lambda g, b: (0, 0, 0)),
            pl.BlockSpec((ncb, 1, _CB), lambda g, b: (0, 0, 0)),
        ],
        out_specs=pl.BlockSpec((_R, _KP), lambda g, b: (g, 0)),
    )
    return pl.pallas_call(
        functools.partial(_knn_body, n_total=n),
        grid_spec=grid_spec,
        out_shape=jax.ShapeDtypeStruct((n, _KP), jnp.int32),
    )(bounds, featb, sqr, brow, xtc, sqc, bcol)


def _gather_rows(table, idx):
    m = idx.shape[0]
    d = table.shape[1]
    info = plsc.get_sparse_core_info()
    nw = info.num_cores * info.num_subcores
    per_w = m // nw
    chunk = 512
    steps = per_w // chunk
    mesh = plsc.VectorSubcoreMesh(core_axis_name="c", subcore_axis_name="s")

    @functools.partial(
        pl.kernel,
        out_type=jax.ShapeDtypeStruct((m, d), jnp.float32),
        mesh=mesh,
        compiler_params=pltpu.CompilerParams(use_tc_tiling_on_sc=False),
        scratch_types=[
            pltpu.VMEM((chunk,), jnp.int32),
            pltpu.VMEM((chunk, d), jnp.float32),
            pltpu.SemaphoreType.DMA,
        ],
    )
    def gk(table_hbm, idx_hbm, out_hbm, idx_v, rows_v, sem):
        wid = lax.axis_index("s") * info.num_cores + lax.axis_index("c")
        base = wid * per_w

        def body(i, carry):
            off = base + i * chunk
            pltpu.sync_copy(idx_hbm.at[pl.ds(off, chunk)], idx_v)
            pltpu.async_copy(table_hbm.at[idx_v], rows_v, sem).wait()
            pltpu.sync_copy(rows_v, out_hbm.at[pl.ds(off, chunk)])
            return carry

        lax.fori_loop(0, steps, body, 0)

    return gk(table, idx)


def _edge_body(x_ref, g_ref, wa_ref, wb_ref, b1_ref, g1_ref, bb1_ref,
               w2_ref, b2_ref, g2_ref, bb2_ref, o_ref):
    x = x_ref[...]
    s = _mm(_bf(x), wa_ref[...])
    b1 = b1_ref[...]
    g1 = g1_ref[...]
    bb1 = bb1_ref[...]
    wb = wb_ref[...]
    w2 = w2_ref[...]
    b2 = b2_ref[...]
    g2 = g2_ref[...]
    bb2 = bb2_ref[...]
    acc = jnp.zeros(o_ref.shape, jnp.float32)
    for j in range(_K):
        t = _mm(_bf(g_ref[j] - x), wb)
        h1 = jnp.maximum(_bn((s + t) + b1, g1, bb1), 0.0)
        h2 = jnp.maximum(_bn(_mm(_bf(h1), w2) + b2, g2, bb2), 0.0)
        acc = jnp.maximum(acc, h2)
    o_ref[...] = acc


def _edge(feat, gath, wa, wb, b1, g1, bb1, w2, b2, g2, bb2):
    n, din = feat.shape
    dout = w2.shape[1]
    full = lambda a: pl.BlockSpec(a.shape, lambda i: (0,) * a.ndim)
    return pl.pallas_call(
        _edge_body,
        grid=(n // _RE,),
        in_specs=[
            pl.BlockSpec((_RE, din), lambda i: (i, 0)),
            pl.BlockSpec((_K, _RE, din), lambda i: (0, i, 0)),
            full(wa), full(wb), full(b1), full(g1), full(bb1),
            full(w2), full(b2), full(g2), full(bb2),
        ],
        out_specs=pl.BlockSpec((_RE, dout), lambda i: (i, 0)),
        out_shape=jax.ShapeDtypeStruct((n, dout), jnp.float32),
    )(feat, gath, wa, wb, b1, g1, bb1, w2, b2, g2, bb2)


def _conv(feat, sq, brow, bcol, bounds, start_r, size_r, p, tag, din_pad):
    n, dp = feat.shape
    w1 = p[tag + 'W1']
    din = w1.shape[1] // 2
    pad = ((0, din_pad - din), (0, 0))
    wa = _bf(jnp.pad(w1[:, :din].T, pad))
    wb = _bf(jnp.pad(w1[:, din:].T, pad))
    idx = _knn(_bf(feat), sq, brow, bcol, bounds)[:, :_K]
    valid = idx < n
    cnt = jnp.sum(valid, axis=1).astype(jnp.int32)
    pfill = jnp.arange(_K, dtype=jnp.int32)[None, :] - cnt[:, None]
    fill = jnp.where(pfill < start_r[:, None], pfill, pfill + size_r[:, None])
    idx = jnp.where(valid, idx, fill)
    idx_t = idx.T.reshape(-1)
    gath = _gather_rows(feat, idx_t).reshape(_K, n, dp)
    row = lambda v: v[None, :]
    return _edge(feat, gath, wa, wb,
                 row(p[tag + 'b1']), row(p[tag + 'g1']), row(p[tag + 'bb1']),
                 _bf(p[tag + 'W2'].T),
                 row(p[tag + 'b2']), row(p[tag + 'g2']), row(p[tag + 'bb2']))


def _head_body(x1_ref, x2_ref, x3_ref, w1a_ref, w1b_ref, w1c_ref, b1_ref,
               g1_ref, bb1_ref, w2_ref, b2_ref, g2_ref, bb2_ref,
               w3_ref, b3_ref, o_ref):
    y = (_mm(_bf(x1_ref[...]), w1a_ref[...])
         + _mm(_bf(x2_ref[...]), w1b_ref[...])
         + _mm(_bf(x3_ref[...]), w1c_ref[...])
         + b1_ref[...])
    z = jnp.maximum(_bn(y, g1_ref[...], bb1_ref[...]), 0.0)
    z = jnp.maximum(
        _bn(_mm(_bf(z), w2_ref[...]) + b2_ref[...], g2_ref[...], bb2_ref[...]),
        0.0)
    o_ref[...] = _mm(_bf(z), w3_ref[...]) + b3_ref[...]


def _head(x1, x2, x3, *ws):
    n = x1.shape[0]
    full = lambda a: pl.BlockSpec(a.shape, lambda i: (0,) * a.ndim)
    return pl.pallas_call(
        _head_body,
        grid=(n // _RH,),
        in_specs=[
            pl.BlockSpec((_RH, x1.shape[1]), lambda i: (i, 0)),
            pl.BlockSpec((_RH, x2.shape[1]), lambda i: (i, 0)),
            pl.BlockSpec((_RH, x3.shape[1]), lambda i: (i, 0)),
        ] + [full(w) for w in ws],
        out_specs=pl.BlockSpec((_RH, 8), lambda i: (i, 0)),
        out_shape=jax.ShapeDtypeStruct((n, 8), jnp.float32),
    )(x1, x2, x3, *ws)


def kernel(x, batch, params):
    p = params
    n, d_in = x.shape
    batch = batch.astype(jnp.int32)

    xp = jnp.pad(x, ((0, 0), (0, 16 - d_in)))
    w1g = _bf(jnp.pad(p['aW1'].T, ((0, 16 - d_in), (0, 0))))
    w2g = _bf(jnp.pad(p['aW2'].T, ((0, 0), (0, 16 - d_in))))
    b2g = jnp.pad(p['ab2'], (0, 16 - d_in))[None, :]
    x0 = _gate(xp, w1g, p['ab1'][None, :], w2g, b2g)

    ii = jnp.arange(n, dtype=jnp.int32)
    first = jnp.concatenate([jnp.ones((1,), jnp.bool_),
                             batch[1:] != batch[:-1]])
    last = jnp.concatenate([batch[1:] != batch[:-1],
                            jnp.ones((1,), jnp.bool_)])
    start_r = lax.associative_scan(jnp.maximum, jnp.where(first, ii, 0))
    end_r = lax.associative_scan(jnp.minimum, jnp.where(last, ii + 1, n),
                                 reverse=True)
    size_r = end_r - start_r
    r0 = jnp.arange(0, n, _R, dtype=jnp.int32)
    lo = start_r[r0]
    hi = end_r[r0 + _R - 1]
    bounds = jnp.stack([lo // _CB, (hi + _CB - 1) // _CB], axis=1)
    brow = batch[:, None]
    bcol = batch.reshape(n // _CB, 1, _CB)

    sq0 = jnp.sum(x0[:, :d_in] * x0[:, :d_in], axis=1)
    x1 = _conv(x0, sq0, brow, bcol, bounds, start_r, size_r, p, 'c1', 16)
    sq1 = jnp.sum(x1 * x1, axis=1)
    x2 = _conv(x1, sq1, brow, bcol, bounds, start_r, size_r, p, 'c2', 64)
    sq2 = jnp.sum(x2 * x2, axis=1)
    x3 = _conv(x2, sq2, brow, bcol, bounds, start_r, size_r, p, 'c3', 128)

    w1t = _bf(p['pW1'].T)
    row = lambda v: v[None, :]
    out = _head(x1, x2, x3,
                w1t[:64], w1t[64:192], w1t[192:],
                row(p['pb1']), row(p['pg1']), row(p['pbb1']),
                _bf(p['pW2'].T), row(p['pb2']), row(p['pg2']), row(p['pbb2']),
                _bf(jnp.pad(p['pW3'].T, ((0, 0), (0, 3)))),
                row(jnp.pad(p['pb3'], (0, 3))))
    return out[:, :5]
